# baseline (speedup 1.0000x reference)
import functools

import jax
import jax.numpy as jnp
from jax import lax
from jax.experimental import pallas as pl
from jax.experimental.pallas import tpu as pltpu
from jax.experimental.pallas import tpu_sc as plsc

MAXLEN = 200
VOCAB = 100000
CPLEN = 200
NCHUNK = 25
CHUNK = VOCAB // NCHUNK


def _build_sc_gather():
    mesh = plsc.VectorSubcoreMesh(core_axis_name="c", subcore_axis_name="s")

    @functools.partial(
        pl.kernel,
        mesh=mesh,
        out_type=(
            jax.ShapeDtypeStruct((NCHUNK, CHUNK), jnp.float32),
            jax.ShapeDtypeStruct((NCHUNK, CHUNK), jnp.float32),
            jax.ShapeDtypeStruct((CPLEN,), jnp.float32),
        ),
        scratch_types=(
            pltpu.VMEM((16,), jnp.int32),
            pltpu.VMEM((CHUNK,), jnp.float32),
            pltpu.VMEM((CHUNK,), jnp.float32),
            pltpu.VMEM((CPLEN,), jnp.float32),
            pltpu.SemaphoreType.DMA,
            pltpu.SemaphoreType.DMA,
            pltpu.SemaphoreType.DMA,
        ),
    )
    def gather_rows(rule_hbm, token_hbm, copy_hbm, len_hbm,
                    rule_out, token_out, copy_out,
                    len_v, rbuf, tbuf, cbuf, sem_r, sem_t, sem_c):
        wid = lax.axis_index("c") * 16 + lax.axis_index("s")

        pltpu.sync_copy(len_hbm.at[pl.ds(0, 16)], len_v)
        lv = len_v[...]
        l0 = lv[0]
        a = l0 - 1
        r = lax.rem(a, MAXLEN)
        l = jnp.where(r < 0, r + MAXLEN, r)

        @pl.when(wid < NCHUNK)
        def _():
            row = l * NCHUNK + wid
            cp_r = pltpu.async_copy(rule_hbm.at[row], rbuf, sem_r)
            cp_t = pltpu.async_copy(token_hbm.at[row], tbuf, sem_t)
            cp_r.wait()
            cp_t.wait()
            pltpu.sync_copy(rbuf, rule_out.at[wid])
            pltpu.sync_copy(tbuf, token_out.at[wid])

        @pl.when(wid == NCHUNK)
        def _():
            pltpu.async_copy(copy_hbm.at[l], cbuf, sem_c).wait()
            pltpu.sync_copy(cbuf, copy_out)

    return gather_rows


_sc_gather = _build_sc_gather()


def kernel(rule_prob, token_prob, copy_prob, length):
    rule_r = rule_prob.reshape(MAXLEN * NCHUNK, CHUNK)
    token_r = token_prob.reshape(MAXLEN * NCHUNK, CHUNK)
    r, t, c = _sc_gather(rule_r, token_r, copy_prob, length)
    return (r.reshape(VOCAB), t.reshape(VOCAB), c, length)


# speedup vs baseline: 4.5538x; 4.5538x over previous
import functools

import jax
import jax.numpy as jnp
from jax import lax
from jax.experimental import pallas as pl
from jax.experimental.pallas import tpu as pltpu
from jax.experimental.pallas import tpu_sc as plsc

MAXLEN = 200
VOCAB = 100000
CPLEN = 200
NCHUNK = 25
CHUNK = VOCAB // NCHUNK


def _build_sc_gather():
    mesh = plsc.VectorSubcoreMesh(core_axis_name="c", subcore_axis_name="s")

    @functools.partial(
        pl.kernel,
        mesh=mesh,
        out_type=(
            jax.ShapeDtypeStruct((VOCAB,), jnp.float32),
            jax.ShapeDtypeStruct((VOCAB,), jnp.float32),
            jax.ShapeDtypeStruct((CPLEN,), jnp.float32),
        ),
        scratch_types=(
            pltpu.VMEM((16,), jnp.int32),
            pltpu.VMEM((VOCAB,), jnp.float32),
            pltpu.VMEM((CPLEN,), jnp.float32),
            pltpu.SemaphoreType.DMA,
        ),
    )
    def gather_rows(rule_hbm, token_hbm, copy_hbm, len_hbm,
                    rule_out, token_out, copy_out,
                    len_v, buf, cbuf, sem):
        wid = lax.axis_index("c") * 16 + lax.axis_index("s")

        pltpu.sync_copy(len_hbm.at[pl.ds(0, 16)], len_v)
        l0 = len_v[...][0]
        a = l0 - 1
        r = lax.rem(a, MAXLEN)
        l = jnp.where(r < 0, r + MAXLEN, r)

        @pl.when(wid == 0)
        def _():
            pltpu.async_copy(rule_hbm.at[l], buf, sem).wait()
            pltpu.sync_copy(buf, rule_out)

        @pl.when(wid == 1)
        def _():
            pltpu.async_copy(token_hbm.at[l], buf, sem).wait()
            pltpu.sync_copy(buf, token_out)

        @pl.when(wid == 2)
        def _():
            pltpu.async_copy(copy_hbm.at[l], cbuf, sem).wait()
            pltpu.sync_copy(cbuf, copy_out)

    return gather_rows


_sc_gather = _build_sc_gather()


def kernel(rule_prob, token_prob, copy_prob, length):
    r, t, c = _sc_gather(rule_prob, token_prob, copy_prob, length)
    return (r, t, c, length)


# speedup vs baseline: 4.9618x; 1.0896x over previous
import functools

import jax
import jax.numpy as jnp
from jax import lax
from jax.experimental import pallas as pl
from jax.experimental.pallas import tpu as pltpu
from jax.experimental.pallas import tpu_sc as plsc

MAXLEN = 200
VOCAB = 100000
CPLEN = 200
NSPLIT = 4
QCHUNK = VOCAB // NSPLIT


def _build_sc_gather():
    mesh = plsc.VectorSubcoreMesh(core_axis_name="c", subcore_axis_name="s")

    @functools.partial(
        pl.kernel,
        mesh=mesh,
        out_type=(
            jax.ShapeDtypeStruct((VOCAB,), jnp.float32),
            jax.ShapeDtypeStruct((VOCAB,), jnp.float32),
            jax.ShapeDtypeStruct((CPLEN,), jnp.float32),
        ),
        scratch_types=(
            pltpu.VMEM((16,), jnp.int32),
            pltpu.VMEM((VOCAB,), jnp.float32),
            pltpu.VMEM((CPLEN,), jnp.float32),
            pltpu.SemaphoreType.DMA,
        ),
    )
    def gather_rows(rule_hbm, token_hbm, copy_hbm, len_hbm,
                    rule_out, token_out, copy_out,
                    len_v, buf, cbuf, sem):
        cid = lax.axis_index("c")
        sid = lax.axis_index("s")

        pltpu.sync_copy(len_hbm.at[pl.ds(0, 16)], len_v)
        l0 = len_v[...][0]
        a = l0 - 1
        r = lax.rem(a, MAXLEN)
        l = jnp.where(r < 0, r + MAXLEN, r)

        @pl.when(sid < NSPLIT)
        def _():
            @pl.when(cid == 0)
            def _():
                pltpu.async_copy(rule_hbm.at[l], buf, sem).wait()
                col = sid * QCHUNK
                pltpu.sync_copy(buf.at[pl.ds(col, QCHUNK)],
                                rule_out.at[pl.ds(col, QCHUNK)])

            @pl.when(cid == 1)
            def _():
                pltpu.async_copy(token_hbm.at[l], buf, sem).wait()
                col = sid * QCHUNK
                pltpu.sync_copy(buf.at[pl.ds(col, QCHUNK)],
                                token_out.at[pl.ds(col, QCHUNK)])

        @pl.when((sid == NSPLIT) & (cid == 0))
        def _():
            pltpu.async_copy(copy_hbm.at[l], cbuf, sem).wait()
            pltpu.sync_copy(cbuf, copy_out)

    return gather_rows


_sc_gather = _build_sc_gather()


def kernel(rule_prob, token_prob, copy_prob, length):
    r, t, c = _sc_gather(rule_prob, token_prob, copy_prob, length)
    return (r, t, c, length)


# speedup vs baseline: 5.3405x; 1.0763x over previous
import functools

import jax
import jax.numpy as jnp
from jax import lax
from jax.experimental import pallas as pl
from jax.experimental.pallas import tpu as pltpu
from jax.experimental.pallas import tpu_sc as plsc

MAXLEN = 200
VOCAB = 100000
CPLEN = 200
NTILE = 16
WCOL = 6272
WLAST = 5888
TAILOFF = (NTILE - 1) * WCOL + WLAST
TAILW = VOCAB - TAILOFF


def _build_sc_gather():
    mesh = plsc.VectorSubcoreMesh(core_axis_name="c", subcore_axis_name="s")

    @functools.partial(
        pl.kernel,
        mesh=mesh,
        out_type=(
            jax.ShapeDtypeStruct((VOCAB,), jnp.float32),
            jax.ShapeDtypeStruct((VOCAB,), jnp.float32),
            jax.ShapeDtypeStruct((CPLEN,), jnp.float32),
        ),
        scratch_types=(
            pltpu.VMEM((16,), jnp.int32),
            pltpu.VMEM((8, WCOL), jnp.float32),
            pltpu.VMEM((CPLEN,), jnp.float32),
            pltpu.VMEM((TAILW,), jnp.float32),
            pltpu.SemaphoreType.DMA,
        ),
    )
    def gather_rows(rule_hbm, token_hbm, copy_hbm, len_hbm,
                    rule_tail_hbm, token_tail_hbm,
                    rule_out, token_out, copy_out,
                    len_v, buf, cbuf, tailbuf, sem):
        cid = lax.axis_index("c")
        sid = lax.axis_index("s")

        pltpu.sync_copy(len_hbm.at[pl.ds(0, 16)], len_v)
        l0 = len_v[...][0]
        a = l0 - 1
        r = lax.rem(a, MAXLEN)
        l = jnp.where(r < 0, r + MAXLEN, r)

        lb = pl.multiple_of((l // 8) * 8, 8)
        lmod = l - lb

        for k, (tab, out) in enumerate(
                ((rule_hbm, rule_out), (token_hbm, token_out))):
            @pl.when(cid == k)
            def _(tab=tab, out=out):
                @pl.when(sid < NTILE - 1)
                def _():
                    col = sid * WCOL
                    pltpu.async_copy(
                        tab.at[pl.ds(lb, 8), pl.ds(col, WCOL)], buf, sem
                    ).wait()
                    pltpu.sync_copy(buf.at[lmod],
                                    out.at[pl.ds(col, WCOL)])

                @pl.when(sid == NTILE - 1)
                def _():
                    col = (NTILE - 1) * WCOL
                    bl = buf.at[:, pl.ds(0, WLAST)]
                    pltpu.async_copy(
                        tab.at[pl.ds(lb, 8), pl.ds(col, WLAST)], bl, sem
                    ).wait()
                    pltpu.sync_copy(buf.at[lmod, pl.ds(0, WLAST)],
                                    out.at[pl.ds(col, WLAST)])
                    tail = (rule_tail_hbm, token_tail_hbm)[k]
                    pltpu.async_copy(tail.at[l], tailbuf, sem).wait()
                    pltpu.sync_copy(tailbuf,
                                    out.at[pl.ds(TAILOFF, TAILW)])

        @pl.when((sid == 0) & (cid == 0))
        def _():
            pltpu.async_copy(copy_hbm.at[l], cbuf, sem).wait()
            pltpu.sync_copy(cbuf, copy_out)

    return gather_rows


_sc_gather = _build_sc_gather()


def kernel(rule_prob, token_prob, copy_prob, length):
    rule_tail = jax.lax.slice(rule_prob, (0, TAILOFF), (MAXLEN, VOCAB))
    token_tail = jax.lax.slice(token_prob, (0, TAILOFF), (MAXLEN, VOCAB))
    r, t, c = _sc_gather(rule_prob, token_prob, copy_prob, length,
                         rule_tail, token_tail)
    return (r, t, c, length)
